# baseline (speedup 1.0000x reference)
import functools

import jax
import jax.numpy as jnp
from jax import lax
from jax.experimental import pallas as pl
from jax.experimental.pallas import tpu as pltpu
from jax.experimental.pallas import tpu_sc as plsc

N = 10000
E = 320000
DIN = 128
HID = 32
DOUT = 128

NP = 10112
WID = 40
NW = 32
BLK = 128
EC = E + N
NB = -(-EC // (NW * BLK))
EPW = NB * BLK
RPT = NP // 16

_f32 = jnp.float32
_i32 = jnp.int32


def _tc_entry_body(x_ref, w0_ref, b0_ref, w1_ref, a1_ref, t_ref, scal_ref):
    h0 = jnp.dot(x_ref[...], w0_ref[...], preferred_element_type=_f32)
    h0 = h0 + b0_ref[...]
    t = jnp.dot(h0, w1_ref[...], preferred_element_type=_f32)
    t_ref[...] = t
    scal_ref[...] = lax.dot_general(
        a1_ref[...], t, (((1,), (1,)), ((), ())), preferred_element_type=_f32)


_SCAL_ROWS = 16


def _tc_combine_body(acc_ref, bp_ref, w_ref, a_ref, t_ref, scal_ref):
    a = acc_ref[0] + acc_ref[1]
    u = a[:, :HID]
    den = a[:, HID:HID + 1] + 1e-16
    h = jnp.maximum(u / den + bp_ref[...], 0.0)
    t = jnp.dot(h, w_ref[...], preferred_element_type=_f32)
    t_ref[...] = t
    scal_ref[...] = lax.dot_general(
        a_ref[...], t, (((1,), (1,)), ((), ())), preferred_element_type=_f32)


def _tc_final_body(acc_ref, bp_ref, w4_ref, b4_ref, o_ref):
    a = acc_ref[0] + acc_ref[1]
    u = a[:, :HID]
    den = a[:, HID:HID + 1] + 1e-16
    h = jnp.maximum(u / den + bp_ref[...], 0.0)
    o_ref[...] = jnp.dot(h, w4_ref[...], preferred_element_type=_f32) + b4_ref[...]


_tc_entry = pl.pallas_call(
    _tc_entry_body,
    out_shape=[jax.ShapeDtypeStruct((NP, HID), _f32),
               jax.ShapeDtypeStruct((_SCAL_ROWS, NP), _f32)],
)

_tc_combine = pl.pallas_call(
    _tc_combine_body,
    out_shape=[jax.ShapeDtypeStruct((NP, HID), _f32),
               jax.ShapeDtypeStruct((_SCAL_ROWS, NP), _f32)],
)

_tc_final = pl.pallas_call(
    _tc_final_body,
    out_shape=jax.ShapeDtypeStruct((NP, DOUT), _f32),
)


def _sc_edge_body(t_hbm, scal_hbm, src_hbm, dst_hbm, acc_out,
                  srcv, dstv, altab, artab, hjb, hib, wb, tbuf,
                  zb, acc_sh, gsem, ssem):
    cid = lax.axis_index("c")
    sid = lax.axis_index("s")
    w = cid * 16 + sid

    def zloop(i, _):
        for jj in range(2):
            zb[i, pl.ds(jj * 16, 16)] = jnp.zeros((16,), _f32)
        zb[i, pl.ds(WID - 16, 16)] = jnp.zeros((16,), _f32)
        wb[i, pl.ds(WID - 16, 16)] = jnp.zeros((16,), _f32)
        wb[i + BLK, pl.ds(WID - 16, 16)] = jnp.zeros((16,), _f32)
        return 0
    lax.fori_loop(0, BLK, zloop, 0)

    nfull = RPT // BLK
    tail = RPT - nfull * BLK
    for k in range(nfull):
        pltpu.sync_copy(zb, acc_sh.at[pl.ds(sid * RPT + k * BLK, BLK)])
    if tail:
        pltpu.sync_copy(zb.at[pl.ds(0, tail)],
                        acc_sh.at[pl.ds(sid * RPT + nfull * BLK, tail)])

    pltpu.sync_copy(scal_hbm.at[0], altab)
    pltpu.sync_copy(scal_hbm.at[8], artab)
    pltpu.sync_copy(src_hbm.at[w], srcv)
    pltpu.sync_copy(dst_hbm.at[w], dstv)

    plsc.subcore_barrier()

    lanes = lax.iota(_i32, 16)

    def issue_gathers(j, p):
        pltpu.async_copy(t_hbm.at[srcv.at[j]], hjb.at[pl.ds(p * BLK, BLK)], gsem)
        pltpu.async_copy(t_hbm.at[dstv.at[j]], hib.at[pl.ds(p * BLK, BLK)], gsem)

    def wait_gathers(j, p):
        pltpu.make_async_copy(
            t_hbm.at[srcv.at[j]], hjb.at[pl.ds(p * BLK, BLK)], gsem).wait()
        pltpu.make_async_copy(
            t_hbm.at[dstv.at[j]], hib.at[pl.ds(p * BLK, BLK)], gsem).wait()

    def wait_scatter(j, p):
        pltpu.make_async_copy(
            wb.at[pl.ds(p * BLK, BLK)], acc_sh.at[dstv.at[j]], ssem).wait()

    issue_gathers(0, 0)

    def blk_body(j, _):
        p = lax.rem(j, 2)
        pb = p * BLK
        wait_gathers(j, p)

        @pl.when(j + 1 < NB)
        def _():
            issue_gathers(j + 1, 1 - p)

        for g in range(BLK // 16):
            gb = pb + g * 16
            cg = g * 17
            arows = []
            for u in range(16):
                er = gb + u
                a0 = hjb[er, pl.ds(0, 16)]
                a1 = hjb[er, pl.ds(16, 16)]
                b0 = hib[er, pl.ds(0, 16)]
                b1 = hib[er, pl.ds(16, 16)]
                arows.append((a0, a1))
                plsc.store_scatter(tbuf, [lanes, jnp.full((16,), cg + u, _i32)],
                                   a0 * b0 + a1 * b1)
            rows = [tbuf[r, pl.ds(cg, 16)] for r in range(16)]
            while len(rows) > 1:
                rows = [rows[i] + rows[i + 1] for i in range(0, len(rows), 2)]
            logit = rows[0]

            sidx = srcv[j, pl.ds(g * 16, 16)]
            didx = dstv[j, pl.ds(g * 16, 16)]
            aj = plsc.load_gather(altab, [sidx])
            ai = plsc.load_gather(artab, [didx])
            alpha = (aj + ai) / (1.0 + jnp.exp(-logit))
            alpha = jnp.where(alpha >= 0.0, alpha, 0.2 * alpha)
            ev = jnp.exp(alpha)

            erows_g = lanes + gb
            plsc.store_scatter(wb, [erows_g, jnp.full((16,), HID, _i32)], ev)
            for u in range(16):
                er = gb + u
                evs = ev[u]
                a0, a1 = arows[u]
                wb[er, pl.ds(0, 16)] = a0 * evs
                wb[er, pl.ds(16, 16)] = a1 * evs

        @pl.when(j > 0)
        def _():
            wait_scatter(j - 1, 1 - p)

        pltpu.async_copy(wb.at[pl.ds(pb, BLK)], acc_sh.at[dstv.at[j]],
                         ssem, add=True)
        return 0

    lax.fori_loop(0, NB, blk_body, 0)
    wait_scatter(NB - 1, (NB - 1) % 2)

    plsc.subcore_barrier()

    for k in range(nfull):
        pltpu.sync_copy(acc_sh.at[pl.ds(sid * RPT + k * BLK, BLK)], zb)
        pltpu.sync_copy(zb, acc_out.at[cid, pl.ds(sid * RPT + k * BLK, BLK)])
    if tail:
        pltpu.sync_copy(acc_sh.at[pl.ds(sid * RPT + nfull * BLK, tail)],
                        zb.at[pl.ds(0, tail)])
        pltpu.sync_copy(zb.at[pl.ds(0, tail)],
                        acc_out.at[cid, pl.ds(sid * RPT + nfull * BLK, tail)])


_sc_edge = functools.partial(
    pl.kernel,
    out_type=jax.ShapeDtypeStruct((2, NP, WID), _f32),
    mesh=plsc.VectorSubcoreMesh(
        core_axis_name="c", subcore_axis_name="s",
        num_cores=2, num_subcores=16),
    compiler_params=pltpu.CompilerParams(
        use_tc_tiling_on_sc=False, needs_layout_passes=False),
    scratch_types=[
        pltpu.VMEM((NB, BLK), _i32),
        pltpu.VMEM((NB, BLK), _i32),
        pltpu.VMEM((NP,), _f32),
        pltpu.VMEM((NP,), _f32),
        pltpu.VMEM((2 * BLK, HID), _f32),
        pltpu.VMEM((2 * BLK, HID), _f32),
        pltpu.VMEM((2 * BLK, WID), _f32),
        pltpu.VMEM((16, 17 * (BLK // 16)), _f32),
        pltpu.VMEM((BLK, WID), _f32),
        pltpu.VMEM_SHARED((NP, WID), _f32),
        pltpu.SemaphoreType.DMA,
        pltpu.SemaphoreType.DMA,
    ],
)(_sc_edge_body)


def _avec(al, ar):
    a = jnp.zeros((_SCAL_ROWS, HID), _f32)
    return a.at[0].set(al).at[8].set(ar)


def kernel(x, edge_index, W0, b0, W1, al1, ar1, b1, W2, al2, ar2, b2,
           W3, al3, ar3, b3, W4, b4):
    xp = jnp.pad(x, ((0, NP - N), (0, 0)))
    loops = jnp.arange(N, dtype=edge_index.dtype)
    src = jnp.concatenate([edge_index[0], loops])
    dst = jnp.concatenate([edge_index[1], loops])
    pad = NW * EPW - EC
    padi = (N + (jnp.arange(pad, dtype=_i32) % 8)).astype(_i32)
    srcp = jnp.concatenate([src, padi]).reshape(NW, NB, BLK)
    dstp = jnp.concatenate([dst, padi]).reshape(NW, NB, BLK)

    t1, scal1 = _tc_entry(xp, W0, b0.reshape(1, HID), W1, _avec(al1, ar1))
    acc1 = _sc_edge(t1, scal1, srcp, dstp)
    t2, scal2 = _tc_combine(acc1, b1.reshape(1, HID), W2, _avec(al2, ar2))
    acc2 = _sc_edge(t2, scal2, srcp, dstp)
    t3, scal3 = _tc_combine(acc2, b2.reshape(1, HID), W3, _avec(al3, ar3))
    acc3 = _sc_edge(t3, scal3, srcp, dstp)
    out = _tc_final(acc3, b3.reshape(1, HID), W4, b4.reshape(1, DOUT))
    return out[:N]


# speedup vs baseline: 1.0371x; 1.0371x over previous
import functools

import jax
import jax.numpy as jnp
from jax import lax
from jax.experimental import pallas as pl
from jax.experimental.pallas import tpu as pltpu
from jax.experimental.pallas import tpu_sc as plsc

N = 10000
E = 320000
DIN = 128
HID = 32
DOUT = 128

NP = 10112
WID = 40
NW = 32
BLK = 128
EC = E + N
NB = -(-EC // (NW * BLK))
EPW = NB * BLK
RPT = NP // 16

_f32 = jnp.float32
_i32 = jnp.int32


def _tc_entry_body(x_ref, w0_ref, b0_ref, w1_ref, a1_ref, t_ref, scal_ref):
    h0 = jnp.dot(x_ref[...], w0_ref[...], preferred_element_type=_f32)
    h0 = h0 + b0_ref[...]
    t = jnp.dot(h0, w1_ref[...], preferred_element_type=_f32)
    t_ref[...] = t
    scal_ref[...] = lax.dot_general(
        a1_ref[...], t, (((1,), (1,)), ((), ())), preferred_element_type=_f32)


_SCAL_ROWS = 16


def _tc_combine_body(acc_ref, bp_ref, w_ref, a_ref, t_ref, scal_ref):
    a = acc_ref[0] + acc_ref[1]
    u = a[:, :HID]
    den = a[:, HID:HID + 1] + 1e-16
    h = jnp.maximum(u / den + bp_ref[...], 0.0)
    t = jnp.dot(h, w_ref[...], preferred_element_type=_f32)
    t_ref[...] = t
    scal_ref[...] = lax.dot_general(
        a_ref[...], t, (((1,), (1,)), ((), ())), preferred_element_type=_f32)


def _tc_final_body(acc_ref, bp_ref, w4_ref, b4_ref, o_ref):
    a = acc_ref[0] + acc_ref[1]
    u = a[:, :HID]
    den = a[:, HID:HID + 1] + 1e-16
    h = jnp.maximum(u / den + bp_ref[...], 0.0)
    o_ref[...] = jnp.dot(h, w4_ref[...], preferred_element_type=_f32) + b4_ref[...]


_tc_entry = pl.pallas_call(
    _tc_entry_body,
    out_shape=[jax.ShapeDtypeStruct((NP, HID), _f32),
               jax.ShapeDtypeStruct((_SCAL_ROWS, NP), _f32)],
)

_tc_combine = pl.pallas_call(
    _tc_combine_body,
    out_shape=[jax.ShapeDtypeStruct((NP, HID), _f32),
               jax.ShapeDtypeStruct((_SCAL_ROWS, NP), _f32)],
)

_tc_final = pl.pallas_call(
    _tc_final_body,
    out_shape=jax.ShapeDtypeStruct((NP, DOUT), _f32),
)


def _sc_edge_body(t_hbm, scal_hbm, src_hbm, dst_hbm, acc_out,
                  srcv, dstv, altab, artab, hjb, hib, wb, tbuf,
                  zb, acc_sh, gsemA, gsemB, ssemA, ssemB):
    cid = lax.axis_index("c")
    sid = lax.axis_index("s")
    w = cid * 16 + sid

    def zloop(i, _):
        for jj in range(2):
            zb[i, pl.ds(jj * 16, 16)] = jnp.zeros((16,), _f32)
        zb[i, pl.ds(WID - 16, 16)] = jnp.zeros((16,), _f32)
        wb[i, pl.ds(WID - 16, 16)] = jnp.zeros((16,), _f32)
        wb[i + BLK, pl.ds(WID - 16, 16)] = jnp.zeros((16,), _f32)
        return 0
    lax.fori_loop(0, BLK, zloop, 0)

    nfull = RPT // BLK
    tail = RPT - nfull * BLK
    for k in range(nfull):
        pltpu.sync_copy(zb, acc_sh.at[pl.ds(sid * RPT + k * BLK, BLK)])
    if tail:
        pltpu.sync_copy(zb.at[pl.ds(0, tail)],
                        acc_sh.at[pl.ds(sid * RPT + nfull * BLK, tail)])

    pltpu.sync_copy(scal_hbm.at[0], altab)
    pltpu.sync_copy(scal_hbm.at[8], artab)
    pltpu.sync_copy(src_hbm.at[w], srcv)
    pltpu.sync_copy(dst_hbm.at[w], dstv)

    plsc.subcore_barrier()

    lanes = lax.iota(_i32, 16)

    def issue_g(j, sem):
        b = lax.rem(j, 3) * BLK
        pltpu.async_copy(t_hbm.at[srcv.at[j]], hjb.at[pl.ds(b, BLK)], sem)
        pltpu.async_copy(t_hbm.at[dstv.at[j]], hib.at[pl.ds(b, BLK)], sem)

    def wait_g(j, sem):
        b = lax.rem(j, 3) * BLK
        pltpu.make_async_copy(
            t_hbm.at[srcv.at[j]], hjb.at[pl.ds(b, BLK)], sem).wait()
        pltpu.make_async_copy(
            t_hbm.at[dstv.at[j]], hib.at[pl.ds(b, BLK)], sem).wait()

    def issue_s(j, sem):
        pltpu.async_copy(wb.at[pl.ds(lax.rem(j, 2) * BLK, BLK)],
                         acc_sh.at[dstv.at[j]], sem, add=True)

    def wait_s(j, sem):
        pltpu.make_async_copy(wb.at[pl.ds(lax.rem(j, 2) * BLK, BLK)],
                              acc_sh.at[dstv.at[j]], sem).wait()

    issue_g(0, gsemA)
    issue_g(1, gsemB)

    def blk_body(j, _):
        p = lax.rem(j, 2)
        pb = p * BLK
        gb3 = lax.rem(j, 3) * BLK

        @pl.when(p == 0)
        def _():
            wait_g(j, gsemA)

        @pl.when(p == 1)
        def _():
            wait_g(j, gsemB)

        @pl.when(jnp.logical_and(j + 2 < NB, p == 0))
        def _():
            issue_g(j + 2, gsemA)

        @pl.when(jnp.logical_and(j + 2 < NB, p == 1))
        def _():
            issue_g(j + 2, gsemB)

        @pl.when(jnp.logical_and(j >= 2, p == 0))
        def _():
            wait_s(j - 2, ssemA)

        @pl.when(jnp.logical_and(j >= 2, p == 1))
        def _():
            wait_s(j - 2, ssemB)

        for g in range(BLK // 16):
            gb = gb3 + g * 16
            wgb = pb + g * 16
            cg = g * 17
            arows = []
            for u in range(16):
                er = gb + u
                a0 = hjb[er, pl.ds(0, 16)]
                a1 = hjb[er, pl.ds(16, 16)]
                b0 = hib[er, pl.ds(0, 16)]
                b1 = hib[er, pl.ds(16, 16)]
                arows.append((a0, a1))
                plsc.store_scatter(tbuf, [lanes, jnp.full((16,), cg + u, _i32)],
                                   a0 * b0 + a1 * b1)
            rows = [tbuf[r, pl.ds(cg, 16)] for r in range(16)]
            while len(rows) > 1:
                rows = [rows[i] + rows[i + 1] for i in range(0, len(rows), 2)]
            logit = rows[0]

            sidx = srcv[j, pl.ds(g * 16, 16)]
            didx = dstv[j, pl.ds(g * 16, 16)]
            aj = plsc.load_gather(altab, [sidx])
            ai = plsc.load_gather(artab, [didx])
            alpha = (aj + ai) / (1.0 + jnp.exp(-logit))
            alpha = jnp.where(alpha >= 0.0, alpha, 0.2 * alpha)
            ev = jnp.exp(alpha)

            erows_g = lanes + wgb
            plsc.store_scatter(wb, [erows_g, jnp.full((16,), HID, _i32)], ev)
            for u in range(16):
                er = wgb + u
                evs = ev[u]
                a0, a1 = arows[u]
                wb[er, pl.ds(0, 16)] = a0 * evs
                wb[er, pl.ds(16, 16)] = a1 * evs

        @pl.when(p == 0)
        def _():
            issue_s(j, ssemA)

        @pl.when(p == 1)
        def _():
            issue_s(j, ssemB)
        return 0

    lax.fori_loop(0, NB, blk_body, 0)
    wait_s(NB - 2, ssemB if (NB - 2) % 2 else ssemA)
    wait_s(NB - 1, ssemB if (NB - 1) % 2 else ssemA)

    plsc.subcore_barrier()

    for k in range(nfull):
        pltpu.sync_copy(acc_sh.at[pl.ds(sid * RPT + k * BLK, BLK)], zb)
        pltpu.sync_copy(zb, acc_out.at[cid, pl.ds(sid * RPT + k * BLK, BLK)])
    if tail:
        pltpu.sync_copy(acc_sh.at[pl.ds(sid * RPT + nfull * BLK, tail)],
                        zb.at[pl.ds(0, tail)])
        pltpu.sync_copy(zb.at[pl.ds(0, tail)],
                        acc_out.at[cid, pl.ds(sid * RPT + nfull * BLK, tail)])


_sc_edge = functools.partial(
    pl.kernel,
    out_type=jax.ShapeDtypeStruct((2, NP, WID), _f32),
    mesh=plsc.VectorSubcoreMesh(
        core_axis_name="c", subcore_axis_name="s",
        num_cores=2, num_subcores=16),
    compiler_params=pltpu.CompilerParams(
        use_tc_tiling_on_sc=False, needs_layout_passes=False),
    scratch_types=[
        pltpu.VMEM((NB, BLK), _i32),
        pltpu.VMEM((NB, BLK), _i32),
        pltpu.VMEM((NP,), _f32),
        pltpu.VMEM((NP,), _f32),
        pltpu.VMEM((3 * BLK, HID), _f32),
        pltpu.VMEM((3 * BLK, HID), _f32),
        pltpu.VMEM((2 * BLK, WID), _f32),
        pltpu.VMEM((16, 17 * (BLK // 16)), _f32),
        pltpu.VMEM((BLK, WID), _f32),
        pltpu.VMEM_SHARED((NP, WID), _f32),
        pltpu.SemaphoreType.DMA,
        pltpu.SemaphoreType.DMA,
        pltpu.SemaphoreType.DMA,
        pltpu.SemaphoreType.DMA,
    ],
)(_sc_edge_body)


def _avec(al, ar):
    a = jnp.zeros((_SCAL_ROWS, HID), _f32)
    return a.at[0].set(al).at[8].set(ar)


def kernel(x, edge_index, W0, b0, W1, al1, ar1, b1, W2, al2, ar2, b2,
           W3, al3, ar3, b3, W4, b4):
    xp = jnp.pad(x, ((0, NP - N), (0, 0)))
    loops = jnp.arange(N, dtype=edge_index.dtype)
    src = jnp.concatenate([edge_index[0], loops])
    dst = jnp.concatenate([edge_index[1], loops])
    pad = NW * EPW - EC
    padi = (N + (jnp.arange(pad, dtype=_i32) % 8)).astype(_i32)
    srcp = jnp.concatenate([src, padi]).reshape(NW, NB, BLK)
    dstp = jnp.concatenate([dst, padi]).reshape(NW, NB, BLK)

    t1, scal1 = _tc_entry(xp, W0, b0.reshape(1, HID), W1, _avec(al1, ar1))
    acc1 = _sc_edge(t1, scal1, srcp, dstp)
    t2, scal2 = _tc_combine(acc1, b1.reshape(1, HID), W2, _avec(al2, ar2))
    acc2 = _sc_edge(t2, scal2, srcp, dstp)
    t3, scal3 = _tc_combine(acc2, b2.reshape(1, HID), W3, _avec(al3, ar3))
    acc3 = _sc_edge(t3, scal3, srcp, dstp)
    out = _tc_final(acc3, b3.reshape(1, HID), W4, b4.reshape(1, DOUT))
    return out[:N]
